# baseline (speedup 1.0000x reference)
import jax
import jax.numpy as jnp
from jax.experimental import pallas as pl


def _pos_body(row_ref, col_ref, out_ref):
    r = row_ref.shape[0]
    nf = row_ref.shape[2]
    w = col_ref.shape[0]
    col = col_ref[...]
    row = row_ref[...]
    out_ref[:, :, 0:nf] = jnp.broadcast_to(col[None, :, :], (r, w, nf))
    out_ref[:, :, nf : 2 * nf] = jnp.broadcast_to(row, (r, w, nf))


def kernel(row_embed, col_embed, bev_h, bev_w):
    h, nf = row_embed.shape
    w, _ = col_embed.shape
    r = 40
    out = pl.pallas_call(
        _pos_body,
        grid=(h // r,),
        in_specs=[
            pl.BlockSpec((r, 1, nf), lambda i: (i, 0, 0)),
            pl.BlockSpec((w, nf), lambda i: (0, 0)),
        ],
        out_specs=pl.BlockSpec((r, w, 2 * nf), lambda i: (i, 0, 0)),
        out_shape=jax.ShapeDtypeStruct((h, w, 2 * nf), jnp.float32),
    )(row_embed.reshape(h, 1, nf), col_embed)
    return out.reshape(1, h * w, 2 * nf)


# speedup vs baseline: 1.0825x; 1.0825x over previous
import jax
import jax.numpy as jnp
from jax.experimental import pallas as pl
from jax.experimental.pallas import tpu as pltpu


def _pos_body(row_ref, col_ref, out_ref):
    r = row_ref.shape[0]
    nf = row_ref.shape[2]
    w = col_ref.shape[0]
    col = col_ref[...]
    row = row_ref[...]
    out_ref[:, :, 0:nf] = jnp.broadcast_to(col[None, :, :], (r, w, nf))
    out_ref[:, :, nf : 2 * nf] = jnp.broadcast_to(row, (r, w, nf))


def kernel(row_embed, col_embed, bev_h, bev_w):
    h, nf = row_embed.shape
    w, _ = col_embed.shape
    r = 20
    out = pl.pallas_call(
        _pos_body,
        grid=(h // r,),
        in_specs=[
            pl.BlockSpec((r, 1, nf), lambda i: (i, 0, 0)),
            pl.BlockSpec((w, nf), lambda i: (0, 0)),
        ],
        out_specs=pl.BlockSpec((r, w, 2 * nf), lambda i: (i, 0, 0)),
        out_shape=jax.ShapeDtypeStruct((h, w, 2 * nf), jnp.float32),
        compiler_params=pltpu.CompilerParams(
            dimension_semantics=("arbitrary",),
        ),
    )(row_embed.reshape(h, 1, nf), col_embed)
    return out.reshape(1, h * w, 2 * nf)
